# Initial kernel scaffold; baseline (speedup 1.0000x reference)
import jax
import jax.numpy as jnp
from jax.experimental import pallas as pl


def kernel(z, fc_w, fc_b, w1, b1, w2, b2, w3, b3, w4, b4, w5, b5, w6, b6, w7, b7):
    raise NotImplementedError("write your pallas kernel here")


# speedup vs baseline: 6.0287x; 6.0287x over previous
import functools

import jax
import jax.numpy as jnp
from jax.experimental import pallas as pl
from jax.experimental.pallas import tpu as pltpu

_VMEM = 64 * 1024 * 1024

_TAPS = {0: ((1, 0), (3, -1)), 1: ((0, 1), (2, 0))}


def _head_kernel(z_ref, wf_ref, bf_ref, w1_ref, b1_ref, o_ref):
    h = jnp.dot(z_ref[...], wf_ref[...], preferred_element_type=jnp.float32)
    h = jnp.maximum(h + bf_ref[...], 0.0)
    y = jnp.dot(h, w1_ref[...], preferred_element_type=jnp.float32)
    o_ref[...] = jnp.maximum(y + b1_ref[...], 0.0)


def _head(z, fc_w, fc_b, w1, b1):
    B, D = z.shape
    Hn = fc_w.shape[1]
    Co = w1.shape[1]
    w1m = w1.reshape(Hn, Co * 16)
    b1v = jnp.repeat(b1, 16).reshape(1, Co * 16)
    y = pl.pallas_call(
        _head_kernel,
        out_shape=jax.ShapeDtypeStruct((B, Co * 16), jnp.float32),
        compiler_params=pltpu.CompilerParams(vmem_limit_bytes=_VMEM),
    )(z, fc_w, fc_b.reshape(1, Hn), w1m, b1v)
    return y.reshape(B, Co, 16).transpose(1, 0, 2).reshape(Co, B * 16)


def _rot(v, k):
    L = v.shape[-1]
    k %= L
    if k == 0:
        return v
    return jnp.concatenate([v[:, k:], v[:, :k]], axis=1)


def _deconv_kernel(x_ref, wm_ref, b_ref, o_ref, *, H, W, Co, act):
    L = x_ref.shape[-1]
    y = jnp.dot(wm_ref[...], x_ref[...],
                preferred_element_type=jnp.float32)
    pos = jax.lax.broadcasted_iota(jnp.int32, (1, L), 1)
    col = pos % W
    row = (pos // W) % H
    rmask = {0: None, -1: row >= 1, 1: row < H - 1}
    cmask = {0: None, -1: col >= 1, 1: col < W - 1}
    bias = b_ref[...]
    for r in (0, 1):
        for c in (0, 1):
            acc = None
            for kh, si in _TAPS[r]:
                for kw, sj in _TAPS[c]:
                    slab = y[(kh * 4 + kw) * Co:(kh * 4 + kw + 1) * Co, :]
                    t = _rot(slab, si * W + sj)
                    m = rmask[si]
                    if cmask[sj] is not None:
                        m = cmask[sj] if m is None else (m & cmask[sj])
                    if m is not None:
                        t = jnp.where(m, t, 0.0)
                    acc = t if acc is None else acc + t
            acc = acc + bias
            if act == "relu":
                acc = jnp.maximum(acc, 0.0)
            elif act == "sigmoid":
                acc = 1.0 / (1.0 + jnp.exp(-acc))
            o_ref[r, c] = acc


def _deconv(x2d, w, b, *, N, H, W, act, steps):
    Ci, Co = w.shape[0], w.shape[1]
    HW = H * W
    wm = jnp.transpose(w, (2, 3, 1, 0)).reshape(16 * Co, Ci)
    G = min(steps, N)
    P = N // G
    L = P * HW
    kern = functools.partial(_deconv_kernel, H=H, W=W, Co=Co, act=act)
    ph = pl.pallas_call(
        kern,
        out_shape=jax.ShapeDtypeStruct((2, 2, Co, N * HW), jnp.float32),
        grid=(G,),
        in_specs=[
            pl.BlockSpec((Ci, L), lambda g: (0, g)),
            pl.BlockSpec((16 * Co, Ci), lambda g: (0, 0)),
            pl.BlockSpec((Co, 1), lambda g: (0, 0)),
        ],
        out_specs=pl.BlockSpec((2, 2, Co, L), lambda g: (0, 0, 0, g)),
        compiler_params=pltpu.CompilerParams(
            dimension_semantics=("parallel",),
            vmem_limit_bytes=_VMEM),
    )(x2d, wm, b.reshape(Co, 1))
    ph = ph.reshape(2, 2, Co, N, H, W)
    return jnp.transpose(ph, (2, 3, 4, 0, 5, 1)).reshape(Co, N * 4 * HW)


_PLAN = (
    ("relu", 2), ("relu", 2), ("relu", 4),
    ("relu", 8), ("none", 16), ("sigmoid", 32),
)


def kernel(z, fc_w, fc_b, w1, b1, w2, b2, w3, b3, w4, b4,
           w5, b5, w6, b6, w7, b7):
    N = z.shape[0]
    x = _head(z, fc_w, fc_b, w1, b1)
    H = W = 4
    for (w, b), (act, steps) in zip(
            ((w2, b2), (w3, b3), (w4, b4), (w5, b5), (w6, b6), (w7, b7)),
            _PLAN):
        x = _deconv(x, w, b, N=N, H=H, W=W, act=act, steps=steps)
        H, W = 2 * H, 2 * W
    Co = w7.shape[1]
    return x.reshape(Co, N, H, W).transpose(1, 0, 2, 3)


# speedup vs baseline: 10.2335x; 1.6975x over previous
import functools

import jax
import jax.numpy as jnp
from jax.experimental import pallas as pl
from jax.experimental.pallas import tpu as pltpu

_VMEM = 64 * 1024 * 1024

_TAPS = {0: ((1, 0), (3, -1)), 1: ((0, 1), (2, 0))}


def _head_kernel(z_ref, wf_ref, bf_ref, w1_ref, b1_ref, o_ref):
    h = jnp.dot(z_ref[...], wf_ref[...], preferred_element_type=jnp.float32)
    h = jnp.maximum(h + bf_ref[...], 0.0)
    y = jnp.dot(h, w1_ref[...], preferred_element_type=jnp.float32)
    o_ref[...] = jnp.maximum(y + b1_ref[...], 0.0)


def _head(z, fc_w, fc_b, w1, b1):
    B, D = z.shape
    Hn = fc_w.shape[1]
    Co = w1.shape[1]
    w1m = w1.reshape(Hn, Co * 16)
    b1v = jnp.repeat(b1, 16).reshape(1, Co * 16)
    y = pl.pallas_call(
        _head_kernel,
        out_shape=jax.ShapeDtypeStruct((B, Co * 16), jnp.float32),
        compiler_params=pltpu.CompilerParams(vmem_limit_bytes=_VMEM),
    )(z, fc_w, fc_b.reshape(1, Hn), w1m, b1v)
    return y.reshape(B, Co, 16).transpose(1, 0, 2).reshape(Co, B * 16)


def _rot(v, k):
    L = v.shape[-1]
    k %= L
    if k == 0:
        return v
    return jnp.concatenate([v[:, k:], v[:, :k]], axis=1)


def _act(v, kind):
    if kind == "relu":
        return jnp.maximum(v, 0.0)
    if kind == "sigmoid":
        return 1.0 / (1.0 + jnp.exp(-v))
    return v


def _shifted(y, Co, kh, kw, di, dj, W, rmask, cmask):
    slab = y[(kh * 4 + kw) * Co:(kh * 4 + kw + 1) * Co, :]
    t = _rot(slab, di * W + dj)
    m = rmask[di]
    if cmask[dj] is not None:
        m = cmask[dj] if m is None else (m & cmask[dj])
    if m is not None:
        t = jnp.where(m, t, 0.0)
    return t


def _dpair_kernel(x_ref, wa_ref, ba_ref, wb_ref, bb_ref, o_ref,
                  *, H, W, CoA, CoB, actA, actB):
    L = x_ref.shape[-1]
    pos = jax.lax.broadcasted_iota(jnp.int32, (1, L), 1)
    col = pos % W
    row = (pos // W) % H
    rmask = {0: None, -1: row >= 1, 1: row < H - 1}
    cmask = {0: None, -1: col >= 1, 1: col < W - 1}

    ya = jnp.dot(wa_ref[...], x_ref[...],
                 preferred_element_type=jnp.float32)
    ba = ba_ref[...]
    pa = {}
    for r in (0, 1):
        for c in (0, 1):
            acc = None
            for kh, si in _TAPS[r]:
                for kw, sj in _TAPS[c]:
                    t = _shifted(ya, CoA, kh, kw, si, sj, W, rmask, cmask)
                    acc = t if acc is None else acc + t
            pa[(r, c)] = _act(acc + ba, actA)

    yb = {q: jnp.dot(wb_ref[...], pa[q], preferred_element_type=jnp.float32)
          for q in pa}
    bb = bb_ref[...]
    for r in (0, 1):
        for c in (0, 1):
            for rho in (0, 1):
                for gam in (0, 1):
                    acc = None
                    for kh, si in _TAPS[r]:
                        for kw, sj in _TAPS[c]:
                            rp, di = (rho + si) % 2, (rho + si) // 2
                            gp, dj = (gam + sj) % 2, (gam + sj) // 2
                            t = _shifted(yb[(rp, gp)], CoB, kh, kw,
                                         di, dj, W, rmask, cmask)
                            acc = t if acc is None else acc + t
                    o_ref[r, c, rho, gam] = _act(acc + bb, actB)


def _dpair(x2d, wA, bA, wB, bB, *, N, H, W, actA, actB, steps, to_nchw):
    CiA, CoA = wA.shape[0], wA.shape[1]
    CoB = wB.shape[1]
    HW = H * W
    wam = jnp.transpose(wA, (2, 3, 1, 0)).reshape(16 * CoA, CiA)
    wbm = jnp.transpose(wB, (2, 3, 1, 0)).reshape(16 * CoB, CoA)
    G = min(steps, N)
    L = (N // G) * HW
    kern = functools.partial(_dpair_kernel, H=H, W=W, CoA=CoA, CoB=CoB,
                             actA=actA, actB=actB)
    ph = pl.pallas_call(
        kern,
        out_shape=jax.ShapeDtypeStruct((2, 2, 2, 2, CoB, N * HW),
                                       jnp.float32),
        grid=(G,),
        in_specs=[
            pl.BlockSpec((CiA, L), lambda g: (0, g)),
            pl.BlockSpec((16 * CoA, CiA), lambda g: (0, 0)),
            pl.BlockSpec((CoA, 1), lambda g: (0, 0)),
            pl.BlockSpec((16 * CoB, CoA), lambda g: (0, 0)),
            pl.BlockSpec((CoB, 1), lambda g: (0, 0)),
        ],
        out_specs=pl.BlockSpec((2, 2, 2, 2, CoB, L),
                               lambda g: (0, 0, 0, 0, 0, g)),
        compiler_params=pltpu.CompilerParams(
            dimension_semantics=("parallel",),
            vmem_limit_bytes=_VMEM),
    )(x2d, wam, bA.reshape(CoA, 1), wbm, bB.reshape(CoB, 1))
    ph = ph.reshape(2, 2, 2, 2, CoB, N, H, W)
    if to_nchw:
        return jnp.transpose(ph, (5, 4, 6, 2, 0, 7, 3, 1)).reshape(
            N, CoB, 4 * H, 4 * W)
    return jnp.transpose(ph, (4, 5, 6, 2, 0, 7, 3, 1)).reshape(
        CoB, N * 16 * HW)


def kernel(z, fc_w, fc_b, w1, b1, w2, b2, w3, b3, w4, b4,
           w5, b5, w6, b6, w7, b7):
    N = z.shape[0]
    x = _head(z, fc_w, fc_b, w1, b1)
    x = _dpair(x, w2, b2, w3, b3, N=N, H=4, W=4,
               actA="relu", actB="relu", steps=2, to_nchw=False)
    x = _dpair(x, w4, b4, w5, b5, N=N, H=16, W=16,
               actA="relu", actB="relu", steps=8, to_nchw=False)
    return _dpair(x, w6, b6, w7, b7, N=N, H=64, W=64,
                  actA="none", actB="sigmoid", steps=16, to_nchw=True)


# speedup vs baseline: 12.8437x; 1.2551x over previous
import functools

import jax
import jax.numpy as jnp
from jax.experimental import pallas as pl
from jax.experimental.pallas import tpu as pltpu

_VMEM = 80 * 1024 * 1024

_TAPS = {0: ((1, 0), (3, -1)), 1: ((0, 1), (2, 0))}


def _head_kernel(z_ref, wf_ref, bf_ref, w1_ref, b1_ref, o_ref):
    h = jnp.dot(z_ref[...], wf_ref[...], preferred_element_type=jnp.float32)
    h = jnp.maximum(h + bf_ref[...], 0.0)
    y = jnp.dot(h, w1_ref[...], preferred_element_type=jnp.float32)
    o_ref[...] = jnp.maximum(y + b1_ref[...], 0.0)


def _head(z, fc_w, fc_b, w1, b1):
    B, D = z.shape
    Hn = fc_w.shape[1]
    Co = w1.shape[1]
    w1m = w1.reshape(Hn, Co * 16)
    b1v = jnp.repeat(b1, 16).reshape(1, Co * 16)
    y = pl.pallas_call(
        _head_kernel,
        out_shape=jax.ShapeDtypeStruct((B, Co * 16), jnp.float32),
        compiler_params=pltpu.CompilerParams(vmem_limit_bytes=_VMEM),
    )(z, fc_w, fc_b.reshape(1, Hn), w1m, b1v)
    return y.reshape(B, Co, 16).transpose(1, 0, 2).reshape(Co, B * 16)


def _rot(v, k):
    L = v.shape[-1]
    k %= L
    if k == 0:
        return v
    return jnp.concatenate([v[:, k:], v[:, :k]], axis=1)


def _act(v, kind):
    if kind == "relu":
        return jnp.maximum(v, 0.0)
    if kind == "sigmoid":
        return 1.0 / (1.0 + jnp.exp(-v))
    return v


def _shifted(y, Co, kh, kw, di, dj, W, rmask, cmask):
    slab = y[(kh * 4 + kw) * Co:(kh * 4 + kw + 1) * Co, :]
    t = _rot(slab, di * W + dj)
    m = rmask[di]
    if cmask[dj] is not None:
        m = cmask[dj] if m is None else (m & cmask[dj])
    if m is not None:
        t = jnp.where(m, t, 0.0)
    return t


def _dchain_kernel(*refs, H, W, specs):
    x_ref = refs[0]
    o_ref = refs[-1]
    L = x_ref.shape[-1]
    pos = jax.lax.broadcasted_iota(jnp.int32, (1, L), 1)
    col = pos % W
    row = (pos // W) % H
    rmask = {0: None, -1: row >= 1, 1: row < H - 1}
    cmask = {0: None, -1: col >= 1, 1: col < W - 1}

    P = {(0, 0): x_ref[...]}
    M = 1
    for li, (Co, act) in enumerate(specs):
        wm = refs[1 + 2 * li][...]
        b = refs[2 + 2 * li][...]
        Y = {k: jnp.dot(wm, P[k], preferred_element_type=jnp.float32)
             for k in P}
        newP = {}
        for mr in range(M):
            for mc in range(M):
                for r in (0, 1):
                    for c in (0, 1):
                        acc = None
                        for kh, si in _TAPS[r]:
                            for kw, sj in _TAPS[c]:
                                src = ((mr + si) % M, (mc + sj) % M)
                                di = (mr + si) // M
                                dj = (mc + sj) // M
                                t = _shifted(Y[src], Co, kh, kw, di, dj,
                                             W, rmask, cmask)
                                acc = t if acc is None else acc + t
                        newP[(2 * mr + r, 2 * mc + c)] = _act(acc + b, act)
        P = newP
        M *= 2
    for (mr, mc), v in P.items():
        o_ref[mr, mc] = v


def _dchain(x2d, wbs, *, N, H, W, acts, steps, to_nchw):
    HW = H * W
    M = 2 ** len(wbs)
    Co = wbs[-1][0].shape[1]
    G = min(steps, N)
    L = (N // G) * HW
    specs = tuple((w.shape[1], a) for (w, _), a in zip(wbs, acts))
    kern = functools.partial(_dchain_kernel, H=H, W=W, specs=specs)

    ops, in_specs = [x2d], [pl.BlockSpec((x2d.shape[0], L), lambda g: (0, g))]
    for w, b in wbs:
        ci, co = w.shape[0], w.shape[1]
        ops.append(jnp.transpose(w, (2, 3, 1, 0)).reshape(16 * co, ci))
        ops.append(b.reshape(co, 1))
        in_specs.append(pl.BlockSpec((16 * co, ci), lambda g: (0, 0)))
        in_specs.append(pl.BlockSpec((co, 1), lambda g: (0, 0)))

    ph = pl.pallas_call(
        kern,
        out_shape=jax.ShapeDtypeStruct((M, M, Co, N * HW), jnp.float32),
        grid=(G,),
        in_specs=in_specs,
        out_specs=pl.BlockSpec((M, M, Co, L), lambda g: (0, 0, 0, g)),
        compiler_params=pltpu.CompilerParams(
            dimension_semantics=("parallel",),
            vmem_limit_bytes=_VMEM),
    )(*ops)
    ph = ph.reshape(M, M, Co, N, H, W)
    if to_nchw:
        return jnp.transpose(ph, (3, 2, 4, 0, 5, 1)).reshape(
            N, Co, M * H, M * W)
    return jnp.transpose(ph, (2, 3, 4, 0, 5, 1)).reshape(Co, N * M * M * HW)


def kernel(z, fc_w, fc_b, w1, b1, w2, b2, w3, b3, w4, b4,
           w5, b5, w6, b6, w7, b7):
    N = z.shape[0]
    x = _head(z, fc_w, fc_b, w1, b1)
    x = _dchain(x, ((w2, b2), (w3, b3), (w4, b4)), N=N, H=4, W=4,
                acts=("relu", "relu", "relu"), steps=2, to_nchw=False)
    return _dchain(x, ((w5, b5), (w6, b6), (w7, b7)), N=N, H=32, W=32,
                   acts=("relu", "none", "sigmoid"), steps=16, to_nchw=True)
